# baseline (speedup 1.0000x reference)
import functools

import jax
import jax.numpy as jnp
from jax import lax
from jax.experimental import pallas as pl
from jax.experimental.pallas import tpu as pltpu
from jax.experimental.pallas import tpu_sc as plsc

N_NODES = 10000
N_EDGES = 320000
D_FEAT = 128
D_HID = 256
D_OUT = 256

NC = 2
NS = 16
NW = NC * NS
CHUNK = 128
CPT = 80
EPT = CPT * CHUNK
E_PAD = NW * EPT
ROWS_PAD = 10240
RPT = ROWS_PAD // NS
DEG_ROWS = ROWS_PAD // 128


def _sc_body(x_hbm, src_hbm, dst_hbm, zeros_hbm, zeros1_hbm,
             aggp_hbm, degp_hbm,
             src_v, dst_v, dstbuf, rows_v0, rows_v1, ones_v, agg_sh, deg_sh,
             sem_g0, sem_g1, sem_d):
    c = lax.axis_index("c")
    s = lax.axis_index("s")
    wid = s * NC + c

    pltpu.sync_copy(zeros_hbm, agg_sh.at[pl.ds(s * RPT, RPT)])
    pltpu.sync_copy(zeros1_hbm.at[pl.ds(s * RPT, RPT)],
                    deg_sh.at[pl.ds(s * RPT, RPT)])

    ones = jnp.ones((16,), jnp.float32)
    for k in range(CHUNK // 16):
        ones_v[pl.ds(k * 16, 16)] = ones

    plsc.subcore_barrier()

    rows = (rows_v0, rows_v1)
    sem_g = (sem_g0, sem_g1)
    base = wid * EPT
    HALF = CPT // 2

    for h in range(2):
        hoff = base + h * HALF * CHUNK
        pltpu.sync_copy(src_hbm.at[pl.ds(hoff, HALF * CHUNK)], src_v)
        pltpu.sync_copy(dst_hbm.at[pl.ds(hoff, HALF * CHUNK)], dst_v)

        def step(i, carry):
            off = i * CHUNK
            for j in range(CHUNK // 16):
                dstbuf[pl.ds(j * 16, 16)] = dst_v[pl.ds(off + j * 16, 16)]
            pltpu.async_copy(x_hbm.at[src_v.at[pl.ds(off, CHUNK)]],
                             rows[0], sem_g[0]).wait()
            pltpu.sync_copy(rows[0], agg_sh.at[dstbuf], add=True)
            pltpu.sync_copy(ones_v, deg_sh.at[dstbuf], add=True)
            return carry

        lax.fori_loop(0, HALF, step, 0)

    plsc.subcore_barrier()

    pltpu.sync_copy(agg_sh.at[pl.ds(s * RPT, RPT)],
                    aggp_hbm.at[c].at[pl.ds(s * RPT, RPT)])
    pltpu.sync_copy(deg_sh.at[pl.ds(s * RPT, RPT)],
                    degp_hbm.at[c].at[pl.ds(s * RPT, RPT)])


def _mlp_body(a0, a1, d0, d1, w1, b1, w2, b2, out):
    a = a0[0] + a1[0]
    d = d0[0] + d1[0]
    a = a / jnp.maximum(d, 1.0)
    h = jnp.dot(a, w1[...], preferred_element_type=jnp.float32) + b1[...]
    h = jnp.maximum(h, 0.0)
    out[...] = jnp.dot(h, w2[...], preferred_element_type=jnp.float32) + b2[...]


def kernel(x, edge_index, W1, b1, W2, b2):
    src = edge_index[0].astype(jnp.int32)
    dst = edge_index[1].astype(jnp.int32)
    pad = E_PAD - N_EDGES
    src = jnp.concatenate([src, jnp.zeros((pad,), jnp.int32)])
    dst_fill = N_NODES + jnp.arange(pad, dtype=jnp.int32) % (ROWS_PAD - N_NODES)
    dst = jnp.concatenate([dst, dst_fill])
    zeros = jnp.zeros((RPT, D_FEAT), jnp.float32)
    zeros1 = jnp.zeros((ROWS_PAD,), jnp.float32)

    mesh = plsc.VectorSubcoreMesh(core_axis_name="c", subcore_axis_name="s",
                                  num_cores=NC, num_subcores=NS)
    sc = pl.kernel(
        _sc_body,
        out_type=(
            jax.ShapeDtypeStruct((NC, ROWS_PAD, D_FEAT), jnp.float32),
            jax.ShapeDtypeStruct((NC, ROWS_PAD), jnp.float32),
        ),
        mesh=mesh,
        scratch_types=[
            pltpu.VMEM((EPT // 2,), jnp.int32),
            pltpu.VMEM((EPT // 2,), jnp.int32),
            pltpu.VMEM((CHUNK,), jnp.int32),
            pltpu.VMEM((CHUNK, D_FEAT), jnp.float32),
            pltpu.VMEM((CHUNK, D_FEAT), jnp.float32),
            pltpu.VMEM((CHUNK,), jnp.float32),
            pltpu.VMEM_SHARED((ROWS_PAD, D_FEAT), jnp.float32),
            pltpu.VMEM_SHARED((ROWS_PAD,), jnp.float32),
            pltpu.SemaphoreType.DMA,
            pltpu.SemaphoreType.DMA,
            pltpu.SemaphoreType.DMA,
        ],
    )
    aggp, degp = sc(x, src, dst, zeros, zeros1)
    degp = degp.reshape(NC, ROWS_PAD, 1)

    R = 400
    grid = (N_NODES // R,)
    out = pl.pallas_call(
        _mlp_body,
        grid=grid,
        in_specs=[
            pl.BlockSpec((1, R, D_FEAT), lambda i: (0, i, 0)),
            pl.BlockSpec((1, R, D_FEAT), lambda i: (1, i, 0)),
            pl.BlockSpec((1, R, 1), lambda i: (0, i, 0)),
            pl.BlockSpec((1, R, 1), lambda i: (1, i, 0)),
            pl.BlockSpec((D_FEAT, D_HID), lambda i: (0, 0)),
            pl.BlockSpec((1, D_HID), lambda i: (0, 0)),
            pl.BlockSpec((D_HID, D_OUT), lambda i: (0, 0)),
            pl.BlockSpec((1, D_OUT), lambda i: (0, 0)),
        ],
        out_specs=pl.BlockSpec((R, D_OUT), lambda i: (i, 0)),
        out_shape=jax.ShapeDtypeStruct((N_NODES, D_OUT), jnp.float32),
    )(aggp, aggp, degp, degp, W1, b1.reshape(1, D_HID), W2,
      b2.reshape(1, D_OUT))
    return out


# speedup vs baseline: 1.1363x; 1.1363x over previous
import functools

import jax
import jax.numpy as jnp
from jax import lax
from jax.experimental import pallas as pl
from jax.experimental.pallas import tpu as pltpu
from jax.experimental.pallas import tpu_sc as plsc

N_NODES = 10000
N_EDGES = 320000
D_FEAT = 128
D_HID = 256
D_OUT = 256

NC = 2
NS = 16
NW = NC * NS
CHUNK = 64
CPT = 160
EPT = CPT * CHUNK
E_PAD = NW * EPT
ROWS_PAD = 10240
RPT = ROWS_PAD // NS


def _sc_body(x_hbm, src_hbm, dst_hbm, zeros_hbm, zeros1_hbm,
             aggp_hbm, degp_hbm,
             src_v, dst_v, rows_v0, rows_v1, ones_v, agg_sh, deg_sh,
             sem_g0, sem_g1, sem_d):
    c = lax.axis_index("c")
    s = lax.axis_index("s")
    wid = s * NC + c

    pltpu.sync_copy(zeros_hbm, agg_sh.at[pl.ds(s * RPT, RPT)])
    pltpu.sync_copy(zeros1_hbm.at[pl.ds(s * RPT, RPT)],
                    deg_sh.at[pl.ds(s * RPT, RPT)])

    ones = jnp.ones((16,), jnp.float32)
    for k in range(CHUNK // 16):
        ones_v[pl.ds(k * 16, 16)] = ones

    base = wid * EPT
    pltpu.sync_copy(src_hbm.at[pl.ds(base, EPT)], src_v)
    pltpu.sync_copy(dst_hbm.at[pl.ds(wid * CPT, CPT)], dst_v)

    plsc.subcore_barrier()

    rows = (rows_v0, rows_v1)
    sem_g = (sem_g0, sem_g1)

    def gather(i, b):
        return pltpu.make_async_copy(
            x_hbm.at[src_v.at[pl.ds(i * CHUNK, CHUNK)]], rows[b], sem_g[b])

    gather(0, 0).start()

    def step(k, carry):
        for b in range(2):
            i = k * 2 + b
            nxt = i + 1

            @pl.when(nxt < CPT)
            def _():
                gather(nxt, (b + 1) % 2).start()

            gather(i, b).wait()
            deg_cp = pltpu.async_copy(ones_v, deg_sh.at[dst_v.at[i]], sem_d,
                                      add=True)
            pltpu.sync_copy(rows[b], agg_sh.at[dst_v.at[i]], add=True)
            deg_cp.wait()
        return carry

    lax.fori_loop(0, CPT // 2, step, 0)

    plsc.subcore_barrier()

    pltpu.sync_copy(agg_sh.at[pl.ds(s * RPT, RPT)],
                    aggp_hbm.at[c].at[pl.ds(s * RPT, RPT)])
    pltpu.sync_copy(deg_sh.at[pl.ds(s * RPT, RPT)],
                    degp_hbm.at[c].at[pl.ds(s * RPT, RPT)])


def _mlp_body(a0, a1, d0, d1, w1, b1, w2, b2, out):
    a = a0[0] + a1[0]
    d = d0[0] + d1[0]
    a = a / jnp.maximum(d, 1.0)
    h = jnp.dot(a, w1[...], preferred_element_type=jnp.float32) + b1[...]
    h = jnp.maximum(h, 0.0)
    out[...] = jnp.dot(h, w2[...], preferred_element_type=jnp.float32) + b2[...]


def kernel(x, edge_index, W1, b1, W2, b2):
    src = edge_index[0].astype(jnp.int32)
    dst = edge_index[1].astype(jnp.int32)
    pad = E_PAD - N_EDGES
    src = jnp.concatenate([src, jnp.zeros((pad,), jnp.int32)])
    dst_fill = N_NODES + jnp.arange(pad, dtype=jnp.int32) % (ROWS_PAD - N_NODES)
    dst = jnp.concatenate([dst, dst_fill]).reshape(E_PAD // CHUNK, CHUNK)
    zeros = jnp.zeros((RPT, D_FEAT), jnp.float32)
    zeros1 = jnp.zeros((ROWS_PAD,), jnp.float32)

    mesh = plsc.VectorSubcoreMesh(core_axis_name="c", subcore_axis_name="s",
                                  num_cores=NC, num_subcores=NS)
    sc = pl.kernel(
        _sc_body,
        out_type=(
            jax.ShapeDtypeStruct((NC, ROWS_PAD, D_FEAT), jnp.float32),
            jax.ShapeDtypeStruct((NC, ROWS_PAD), jnp.float32),
        ),
        mesh=mesh,
        scratch_types=[
            pltpu.VMEM((EPT,), jnp.int32),
            pltpu.VMEM((CPT, CHUNK), jnp.int32),
            pltpu.VMEM((CHUNK, D_FEAT), jnp.float32),
            pltpu.VMEM((CHUNK, D_FEAT), jnp.float32),
            pltpu.VMEM((CHUNK,), jnp.float32),
            pltpu.VMEM_SHARED((ROWS_PAD, D_FEAT), jnp.float32),
            pltpu.VMEM_SHARED((ROWS_PAD,), jnp.float32),
            pltpu.SemaphoreType.DMA,
            pltpu.SemaphoreType.DMA,
            pltpu.SemaphoreType.DMA,
        ],
    )
    aggp, degp = sc(x, src, dst, zeros, zeros1)
    degp = degp.reshape(NC, ROWS_PAD, 1)

    R = 400
    grid = (N_NODES // R,)
    out = pl.pallas_call(
        _mlp_body,
        grid=grid,
        in_specs=[
            pl.BlockSpec((1, R, D_FEAT), lambda i: (0, i, 0)),
            pl.BlockSpec((1, R, D_FEAT), lambda i: (1, i, 0)),
            pl.BlockSpec((1, R, 1), lambda i: (0, i, 0)),
            pl.BlockSpec((1, R, 1), lambda i: (1, i, 0)),
            pl.BlockSpec((D_FEAT, D_HID), lambda i: (0, 0)),
            pl.BlockSpec((1, D_HID), lambda i: (0, 0)),
            pl.BlockSpec((D_HID, D_OUT), lambda i: (0, 0)),
            pl.BlockSpec((1, D_OUT), lambda i: (0, 0)),
        ],
        out_specs=pl.BlockSpec((R, D_OUT), lambda i: (i, 0)),
        out_shape=jax.ShapeDtypeStruct((N_NODES, D_OUT), jnp.float32),
    )(aggp, aggp, degp, degp, W1, b1.reshape(1, D_HID), W2,
      b2.reshape(1, D_OUT))
    return out
